# baseline (speedup 1.0000x reference)
import functools
import jax
import jax.numpy as jnp
from jax import lax
from jax.experimental import pallas as pl
from jax.experimental.pallas import tpu as pltpu
from jax.experimental.pallas import tpu_sc as plsc

N = 10000
D = 128
DO = 2
DOP = 16
E = 320000
NPAD = 10240
NC = 2
NS = 16
NW = NC * NS
RPT = NPAD // NS
BLK = 1024
GRID = NPAD // BLK

CH8 = 80
NCH8 = 4352
K0 = 192
K1 = 72
PH8 = 96
EPAD8 = NCH8 * CH8

CH16 = 128
NCH16 = 2560
K16 = NCH16 // NW
EPAD16 = NCH16 * CH16

_SC_PARAMS = pltpu.CompilerParams(use_tc_tiling_on_sc=False)


def _mesh():
    return plsc.VectorSubcoreMesh(core_axis_name="c", subcore_axis_name="s")


@functools.partial(
    pl.kernel,
    out_type=jax.ShapeDtypeStruct((NPAD,), jnp.float32),
    mesh=_mesh(),
    compiler_params=_SC_PARAMS,
    scratch_types=[
        pltpu.VMEM((NCH16 // NS, CH16), jnp.int32),
        pltpu.VMEM((CH16,), jnp.float32),
        pltpu.VMEM((RPT,), jnp.float32),
        pltpu.VMEM_SHARED((NPAD,), jnp.float32),
        pltpu.SemaphoreType.DMA,
    ],
)
def _deg_sc(dste, deg_out, didx, ones, zbuf, dacc, dsem):
    c = lax.axis_index("c")
    s = lax.axis_index("s")
    nch = NCH16 // NS
    for k in range(CH16 // 16):
        ones[pl.ds(k * 16, 16)] = jnp.ones((16,), jnp.float32)
    for k in range(RPT // 16):
        zbuf[pl.ds(k * 16, 16)] = jnp.zeros((16,), jnp.float32)

    @pl.when(c == 0)
    def _():
        pltpu.sync_copy(zbuf, dacc.at[pl.ds(s * RPT, RPT)])
        plsc.subcore_barrier()
        pltpu.sync_copy(dste.at[pl.ds(s * nch, nch)], didx)

        def body(p, _):
            for b in range(8):
                pltpu.async_copy(ones, dacc.at[didx.at[p * 8 + b]],
                                 dsem, add=True)
            for b in range(8):
                pltpu.make_async_copy(ones, dacc.at[didx.at[p * 8 + b]],
                                      dsem).wait()
            return 0

        lax.fori_loop(0, nch // 8, body, 0)
        plsc.subcore_barrier()
        pltpu.sync_copy(dacc.at[pl.ds(s * RPT, RPT)],
                        deg_out.at[pl.ds(s * RPT, RPT)])


def _run_phase(y, srce, dste, sidx, didx, gbuf, acc, g0, g1, s0, s1,
               base, nch):
    idxcap = sidx.shape[0]
    gsem = (g0, g1)
    ssem = (s0, s1)

    @pl.when(nch > 0)
    def _():
        pltpu.sync_copy(srce.at[pl.ds(base, idxcap)], sidx)
        pltpu.sync_copy(dste.at[pl.ds(base, idxcap)], didx)

        def wait_g(j, b):
            pltpu.make_async_copy(y.at[sidx.at[j]], gbuf.at[b],
                                  gsem[b]).wait()

        def fire_g(j, b):
            pltpu.async_copy(y.at[sidx.at[j]], gbuf.at[b], gsem[b])

        def wait_s(j, b):
            pltpu.make_async_copy(gbuf.at[b], acc.at[didx.at[j]],
                                  ssem[b]).wait()

        def fire_s(j, b):
            pltpu.async_copy(gbuf.at[b], acc.at[didx.at[j]], ssem[b],
                             add=True)

        fire_g(0, 0)
        npairs = nch // 2

        def grp(p, _):
            j0 = p * 2
            j1 = j0 + 1
            wait_g(j0, 0)
            fire_s(j0, 0)

            @pl.when(p > 0)
            def _():
                wait_s(j0 - 1, 1)

            fire_g(j1, 1)
            wait_g(j1, 1)
            fire_s(j1, 1)
            wait_s(j0, 0)

            @pl.when(p < npairs - 1)
            def _():
                fire_g(j0 + 2, 0)

            return 0

        lax.fori_loop(0, npairs, grp, 0)
        wait_s(nch - 1, 1)


def _zero_acc(gbuf, acc, s, width):
    ch = gbuf.shape[1]

    def zrow(r, _):
        for k in range(width // 16):
            gbuf[0, r, pl.ds(k * 16, 16)] = jnp.zeros((16,), jnp.float32)
        return 0

    lax.fori_loop(0, ch, zrow, 0)
    for j in range(RPT // ch):
        pltpu.sync_copy(gbuf.at[0], acc.at[pl.ds(s * RPT + j * ch, ch)])


@functools.partial(
    pl.kernel,
    out_type=jax.ShapeDtypeStruct((NC, NPAD, D), jnp.float32),
    mesh=_mesh(),
    compiler_params=_SC_PARAMS,
    scratch_types=[
        pltpu.VMEM((PH8, CH8), jnp.int32),
        pltpu.VMEM((PH8, CH8), jnp.int32),
        pltpu.VMEM((2, CH8, D), jnp.float32),
        pltpu.VMEM_SHARED((NPAD, D), jnp.float32),
        pltpu.SemaphoreType.DMA,
        pltpu.SemaphoreType.DMA,
        pltpu.SemaphoreType.DMA,
        pltpu.SemaphoreType.DMA,
    ],
)
def _agg128_sc(y1, srce, dste, out, sidx, didx, gbuf, acc, g0, g1, s0, s1):
    c = lax.axis_index("c")
    s = lax.axis_index("s")
    _zero_acc(gbuf, acc, s, D)
    plsc.subcore_barrier()
    base = jnp.where(c == 0, s * K0, NS * K0 + s * K1)
    n0 = jnp.where(c == 0, PH8, K1)
    n1 = jnp.where(c == 0, PH8, 0)
    _run_phase(y1, srce, dste, sidx, didx, gbuf, acc, g0, g1, s0, s1,
               base, n0)
    _run_phase(y1, srce, dste, sidx, didx, gbuf, acc, g0, g1, s0, s1,
               base + PH8, n1)
    plsc.subcore_barrier()
    pltpu.sync_copy(acc.at[pl.ds(s * RPT, RPT)],
                    out.at[c, pl.ds(s * RPT, RPT)])


@functools.partial(
    pl.kernel,
    out_type=jax.ShapeDtypeStruct((NC, NPAD, DOP), jnp.float32),
    mesh=_mesh(),
    compiler_params=_SC_PARAMS,
    scratch_types=[
        pltpu.VMEM((K16, CH16), jnp.int32),
        pltpu.VMEM((K16, CH16), jnp.int32),
        pltpu.VMEM((2, CH16, DOP), jnp.float32),
        pltpu.VMEM_SHARED((NPAD, DOP), jnp.float32),
        pltpu.SemaphoreType.DMA,
        pltpu.SemaphoreType.DMA,
        pltpu.SemaphoreType.DMA,
        pltpu.SemaphoreType.DMA,
    ],
)
def _agg16_sc(y2, srce, dste, out, sidx, didx, gbuf, acc, g0, g1, s0, s1):
    c = lax.axis_index("c")
    s = lax.axis_index("s")
    w = s * NC + c
    _zero_acc(gbuf, acc, s, DOP)
    plsc.subcore_barrier()
    _run_phase(y2, srce, dste, sidx, didx, gbuf, acc, g0, g1, s0, s1,
               w * K16, jnp.int32(K16))
    plsc.subcore_barrier()
    pltpu.sync_copy(acc.at[pl.ds(s * RPT, RPT)],
                    out.at[c, pl.ds(s * RPT, RPT)])


def _tc1_body(deg_ref, x_ref, w1_ref, y1_ref):
    dis = lax.rsqrt(deg_ref[...] + 1.0)
    xw = jnp.dot(x_ref[...], w1_ref[...], preferred_element_type=jnp.float32)
    y1_ref[...] = xw * dis


def _tc2_body(deg_ref, s1_ref, y1_ref, b1_ref, w2_ref, y2_ref):
    dis = lax.rsqrt(deg_ref[...] + 1.0)
    h = (s1_ref[0] + s1_ref[1] + y1_ref[...]) * dis + b1_ref[...]
    h = jnp.maximum(h, 0.0)
    y2_ref[...] = jnp.dot(h, w2_ref[...], preferred_element_type=jnp.float32) * dis


def _tc3_body(deg_ref, s2_ref, y2_ref, b2_ref, o_ref):
    dis = lax.rsqrt(deg_ref[...] + 1.0)
    o_ref[...] = (s2_ref[0] + s2_ref[1] + y2_ref[...]) * dis + b2_ref[...]


def _pad_edges(col, total):
    pad = jnp.full((total - E,), N, dtype=jnp.int32)
    return jnp.concatenate([col, pad])


def kernel(x, edge_index, W1, b1, W2, b2):
    ei = edge_index.astype(jnp.int32)
    srce8 = _pad_edges(ei[0], EPAD8).reshape(NCH8, CH8)
    dste8 = _pad_edges(ei[1], EPAD8).reshape(NCH8, CH8)
    srce16 = _pad_edges(ei[0], EPAD16).reshape(NCH16, CH16)
    dste16 = _pad_edges(ei[1], EPAD16).reshape(NCH16, CH16)
    xp = jnp.pad(x, ((0, NPAD - N), (0, 0)))
    w2p = jnp.pad(W2, ((0, 0), (0, DOP - DO)))
    b1r = b1.reshape(1, D)
    b2r = jnp.pad(b2, (0, DOP - DO)).reshape(1, DOP)

    deg = _deg_sc(dste16).reshape(NPAD, 1)

    y1 = pl.pallas_call(
        _tc1_body,
        grid=(GRID,),
        in_specs=[
            pl.BlockSpec((BLK, 1), lambda i: (i, 0)),
            pl.BlockSpec((BLK, D), lambda i: (i, 0)),
            pl.BlockSpec((D, D), lambda i: (0, 0)),
        ],
        out_specs=pl.BlockSpec((BLK, D), lambda i: (i, 0)),
        out_shape=jax.ShapeDtypeStruct((NPAD, D), jnp.float32),
    )(deg, xp, W1)

    s1 = _agg128_sc(y1, srce8, dste8)

    y2 = pl.pallas_call(
        _tc2_body,
        grid=(GRID,),
        in_specs=[
            pl.BlockSpec((BLK, 1), lambda i: (i, 0)),
            pl.BlockSpec((NC, BLK, D), lambda i: (0, i, 0)),
            pl.BlockSpec((BLK, D), lambda i: (i, 0)),
            pl.BlockSpec((1, D), lambda i: (0, 0)),
            pl.BlockSpec((D, DOP), lambda i: (0, 0)),
        ],
        out_specs=pl.BlockSpec((BLK, DOP), lambda i: (i, 0)),
        out_shape=jax.ShapeDtypeStruct((NPAD, DOP), jnp.float32),
    )(deg, s1, y1, b1r, w2p)

    s2 = _agg16_sc(y2, srce16, dste16)

    o = pl.pallas_call(
        _tc3_body,
        grid=(GRID,),
        in_specs=[
            pl.BlockSpec((BLK, 1), lambda i: (i, 0)),
            pl.BlockSpec((NC, BLK, DOP), lambda i: (0, i, 0)),
            pl.BlockSpec((BLK, DOP), lambda i: (i, 0)),
            pl.BlockSpec((1, DOP), lambda i: (0, 0)),
        ],
        out_specs=pl.BlockSpec((BLK, DOP), lambda i: (i, 0)),
        out_shape=jax.ShapeDtypeStruct((NPAD, DOP), jnp.float32),
    )(deg, s2, y2, b2r)

    return o[:N, :DO]


# speedup vs baseline: 3.0079x; 3.0079x over previous
import functools
import jax
import jax.numpy as jnp
from jax import lax
from jax.experimental import pallas as pl
from jax.experimental.pallas import tpu as pltpu
from jax.experimental.pallas import tpu_sc as plsc

N = 10000
D = 128
DO = 2
DOP = 16
E = 320000
NPAD = 10240
NC = 2
NS = 16
NW = NC * NS
RPT = NPAD // NS
BLK = 1024
GRID = NPAD // BLK
EPAD = 327680

CH8 = 80
K8 = 128
CH16 = 128
K16 = 80
NCH16 = 2560

_SC_PARAMS = pltpu.CompilerParams(use_tc_tiling_on_sc=False)


def _mesh():
    return plsc.VectorSubcoreMesh(core_axis_name="c", subcore_axis_name="s")


@functools.partial(
    pl.kernel,
    out_type=jax.ShapeDtypeStruct((NPAD,), jnp.float32),
    mesh=_mesh(),
    compiler_params=_SC_PARAMS,
    scratch_types=[
        pltpu.VMEM((NCH16 // NS, CH16), jnp.int32),
        pltpu.VMEM((CH16,), jnp.float32),
        pltpu.VMEM((RPT,), jnp.float32),
        pltpu.VMEM_SHARED((NPAD,), jnp.float32),
        pltpu.SemaphoreType.DMA,
    ],
)
def _deg_sc(dste, deg_out, didx, ones, zbuf, dacc, dsem):
    c = lax.axis_index("c")
    s = lax.axis_index("s")
    nch = NCH16 // NS
    for k in range(CH16 // 16):
        ones[pl.ds(k * 16, 16)] = jnp.ones((16,), jnp.float32)
    for k in range(RPT // 16):
        zbuf[pl.ds(k * 16, 16)] = jnp.zeros((16,), jnp.float32)

    @pl.when(c == 0)
    def _():
        pltpu.sync_copy(zbuf, dacc.at[pl.ds(s * RPT, RPT)])
        plsc.subcore_barrier()
        pltpu.sync_copy(dste.at[pl.ds(s * nch, nch)], didx)

        def body(p, _):
            for b in range(8):
                pltpu.async_copy(ones, dacc.at[didx.at[p * 8 + b]],
                                 dsem, add=True)
            for b in range(8):
                pltpu.make_async_copy(ones, dacc.at[didx.at[p * 8 + b]],
                                      dsem).wait()
            return 0

        lax.fori_loop(0, nch // 8, body, 0)
        plsc.subcore_barrier()
        pltpu.sync_copy(dacc.at[pl.ds(s * RPT, RPT)],
                        deg_out.at[pl.ds(s * RPT, RPT)])


def _agg_body(y, srce, dste, out, sidx, didx, gbuf, acc,
              g0, g1, s0, s1, width, nch):
    c = lax.axis_index("c")
    s = lax.axis_index("s")
    w = s * NC + c
    ch = gbuf.shape[1]
    npairs = nch // 2
    gsem = (g0, g1)
    ssem = (s0, s1)

    def zrow(r, _):
        for k in range(width // 16):
            gbuf[0, r, pl.ds(k * 16, 16)] = jnp.zeros((16,), jnp.float32)
        return 0

    lax.fori_loop(0, ch, zrow, 0)
    for j in range(RPT // ch):
        pltpu.sync_copy(gbuf.at[0], acc.at[pl.ds(s * RPT + j * ch, ch)])
    pltpu.sync_copy(srce.at[w], sidx)
    pltpu.sync_copy(dste.at[w], didx)
    plsc.subcore_barrier()

    def wait_g(j, b):
        pltpu.make_async_copy(y.at[sidx.at[j]], gbuf.at[b], gsem[b]).wait()

    def fire_g(j, b):
        pltpu.async_copy(y.at[sidx.at[j]], gbuf.at[b], gsem[b])

    def wait_s(j, b):
        pltpu.make_async_copy(gbuf.at[b], acc.at[didx.at[j]], ssem[b]).wait()

    def fire_s(j, b):
        pltpu.async_copy(gbuf.at[b], acc.at[didx.at[j]], ssem[b], add=True)

    fire_g(0, 0)

    def grp(p, _):
        j0 = p * 2
        j1 = j0 + 1
        wait_g(j0, 0)
        fire_s(j0, 0)

        @pl.when(p > 0)
        def _():
            wait_s(j0 - 1, 1)

        fire_g(j1, 1)
        wait_g(j1, 1)
        fire_s(j1, 1)
        wait_s(j0, 0)

        @pl.when(p < npairs - 1)
        def _():
            fire_g(j0 + 2, 0)

        return 0

    lax.fori_loop(0, npairs, grp, 0)
    wait_s(nch - 1, 1)
    plsc.subcore_barrier()
    pltpu.sync_copy(acc.at[pl.ds(s * RPT, RPT)],
                    out.at[c, pl.ds(s * RPT, RPT)])


@functools.partial(
    pl.kernel,
    out_type=jax.ShapeDtypeStruct((NC, NPAD, D), jnp.float32),
    mesh=_mesh(),
    compiler_params=_SC_PARAMS,
    scratch_types=[
        pltpu.VMEM((K8, CH8), jnp.int32),
        pltpu.VMEM((K8, CH8), jnp.int32),
        pltpu.VMEM((2, CH8, D), jnp.float32),
        pltpu.VMEM_SHARED((NPAD, D), jnp.float32),
        pltpu.SemaphoreType.DMA,
        pltpu.SemaphoreType.DMA,
        pltpu.SemaphoreType.DMA,
        pltpu.SemaphoreType.DMA,
    ],
)
def _agg128_sc(y1, srce, dste, out, sidx, didx, gbuf, acc, g0, g1, s0, s1):
    _agg_body(y1, srce, dste, out, sidx, didx, gbuf, acc,
              g0, g1, s0, s1, D, K8)


@functools.partial(
    pl.kernel,
    out_type=jax.ShapeDtypeStruct((NC, NPAD, DOP), jnp.float32),
    mesh=_mesh(),
    compiler_params=_SC_PARAMS,
    scratch_types=[
        pltpu.VMEM((K16, CH16), jnp.int32),
        pltpu.VMEM((K16, CH16), jnp.int32),
        pltpu.VMEM((2, CH16, DOP), jnp.float32),
        pltpu.VMEM_SHARED((NPAD, DOP), jnp.float32),
        pltpu.SemaphoreType.DMA,
        pltpu.SemaphoreType.DMA,
        pltpu.SemaphoreType.DMA,
        pltpu.SemaphoreType.DMA,
    ],
)
def _agg16_sc(y2, srce, dste, out, sidx, didx, gbuf, acc, g0, g1, s0, s1):
    _agg_body(y2, srce, dste, out, sidx, didx, gbuf, acc,
              g0, g1, s0, s1, DOP, K16)


def _tc1_body(deg_ref, x_ref, w1_ref, y1_ref):
    dis = lax.rsqrt(deg_ref[...] + 1.0)
    xw = jnp.dot(x_ref[...], w1_ref[...], preferred_element_type=jnp.float32)
    y1_ref[...] = xw * dis


def _tc2_body(deg_ref, s1_ref, y1_ref, b1_ref, w2_ref, y2_ref):
    dis = lax.rsqrt(deg_ref[...] + 1.0)
    h = (s1_ref[0] + s1_ref[1] + y1_ref[...]) * dis + b1_ref[...]
    h = jnp.maximum(h, 0.0)
    y2_ref[...] = jnp.dot(h, w2_ref[...], preferred_element_type=jnp.float32) * dis


def _tc3_body(deg_ref, s2_ref, y2_ref, b2_ref, o_ref):
    dis = lax.rsqrt(deg_ref[...] + 1.0)
    o_ref[...] = (s2_ref[0] + s2_ref[1] + y2_ref[...]) * dis + b2_ref[...]


def _pad_edges(col):
    k = EPAD - E
    pad = N + (jnp.arange(k, dtype=jnp.int32) % (NPAD - N))
    return jnp.concatenate([col, pad])


def kernel(x, edge_index, W1, b1, W2, b2):
    ei = edge_index.astype(jnp.int32)
    srcp = _pad_edges(ei[0])
    dstp = _pad_edges(ei[1])
    srce8 = srcp.reshape(NW, K8, CH8)
    dste8 = dstp.reshape(NW, K8, CH8)
    srce16 = srcp.reshape(NW, K16, CH16)
    dste16 = dstp.reshape(NW, K16, CH16)
    xp = jnp.pad(x, ((0, NPAD - N), (0, 0)))
    w2p = jnp.pad(W2, ((0, 0), (0, DOP - DO)))
    b1r = b1.reshape(1, D)
    b2r = jnp.pad(b2, (0, DOP - DO)).reshape(1, DOP)

    deg = _deg_sc(dstp.reshape(NCH16, CH16)).reshape(NPAD, 1)

    y1 = pl.pallas_call(
        _tc1_body,
        grid=(GRID,),
        in_specs=[
            pl.BlockSpec((BLK, 1), lambda i: (i, 0)),
            pl.BlockSpec((BLK, D), lambda i: (i, 0)),
            pl.BlockSpec((D, D), lambda i: (0, 0)),
        ],
        out_specs=pl.BlockSpec((BLK, D), lambda i: (i, 0)),
        out_shape=jax.ShapeDtypeStruct((NPAD, D), jnp.float32),
    )(deg, xp, W1)

    s1 = _agg128_sc(y1, srce8, dste8)

    y2 = pl.pallas_call(
        _tc2_body,
        grid=(GRID,),
        in_specs=[
            pl.BlockSpec((BLK, 1), lambda i: (i, 0)),
            pl.BlockSpec((NC, BLK, D), lambda i: (0, i, 0)),
            pl.BlockSpec((BLK, D), lambda i: (i, 0)),
            pl.BlockSpec((1, D), lambda i: (0, 0)),
            pl.BlockSpec((D, DOP), lambda i: (0, 0)),
        ],
        out_specs=pl.BlockSpec((BLK, DOP), lambda i: (i, 0)),
        out_shape=jax.ShapeDtypeStruct((NPAD, DOP), jnp.float32),
    )(deg, s1, y1, b1r, w2p)

    s2 = _agg16_sc(y2, srce16, dste16)

    o = pl.pallas_call(
        _tc3_body,
        grid=(GRID,),
        in_specs=[
            pl.BlockSpec((BLK, 1), lambda i: (i, 0)),
            pl.BlockSpec((NC, BLK, DOP), lambda i: (0, i, 0)),
            pl.BlockSpec((BLK, DOP), lambda i: (i, 0)),
            pl.BlockSpec((1, DOP), lambda i: (0, 0)),
        ],
        out_specs=pl.BlockSpec((BLK, DOP), lambda i: (i, 0)),
        out_shape=jax.ShapeDtypeStruct((NPAD, DOP), jnp.float32),
    )(deg, s2, y2, b2r)

    return o[:N, :DO]


# speedup vs baseline: 3.4821x; 1.1577x over previous
import functools
import jax
import jax.numpy as jnp
from jax import lax
from jax.experimental import pallas as pl
from jax.experimental.pallas import tpu as pltpu
from jax.experimental.pallas import tpu_sc as plsc

N = 10000
D = 128
DO = 2
DOP = 16
E = 320000
NPAD = 10240
NC = 2
NS = 16
NW = NC * NS
RPT = NPAD // NS
BLK = 1024
GRID = NPAD // BLK
EPAD = 327680

CH8 = 80
K8 = 128
CH16 = 128
K16 = 80
NCH16 = 2560

_SC_PARAMS = pltpu.CompilerParams(use_tc_tiling_on_sc=False)


def _mesh():
    return plsc.VectorSubcoreMesh(core_axis_name="c", subcore_axis_name="s")


@functools.partial(
    pl.kernel,
    out_type=jax.ShapeDtypeStruct((NPAD,), jnp.float32),
    mesh=_mesh(),
    compiler_params=_SC_PARAMS,
    scratch_types=[
        pltpu.VMEM((NCH16 // NS, CH16), jnp.int32),
        pltpu.VMEM((CH16,), jnp.float32),
        pltpu.VMEM((RPT,), jnp.float32),
        pltpu.VMEM_SHARED((NPAD,), jnp.float32),
        pltpu.SemaphoreType.DMA,
    ],
)
def _deg_sc(dste, deg_out, didx, ones, zbuf, dacc, dsem):
    c = lax.axis_index("c")
    s = lax.axis_index("s")
    nch = NCH16 // NS
    for k in range(CH16 // 16):
        ones[pl.ds(k * 16, 16)] = jnp.ones((16,), jnp.float32)
    for k in range(RPT // 16):
        zbuf[pl.ds(k * 16, 16)] = jnp.zeros((16,), jnp.float32)

    @pl.when(c == 0)
    def _():
        pltpu.sync_copy(zbuf, dacc.at[pl.ds(s * RPT, RPT)])
        plsc.subcore_barrier()
        pltpu.sync_copy(dste.at[pl.ds(s * nch, nch)], didx)

        def body(p, _):
            for b in range(8):
                pltpu.async_copy(ones, dacc.at[didx.at[p * 8 + b]],
                                 dsem, add=True)
            for b in range(8):
                pltpu.make_async_copy(ones, dacc.at[didx.at[p * 8 + b]],
                                      dsem).wait()
            return 0

        lax.fori_loop(0, nch // 8, body, 0)
        plsc.subcore_barrier()
        pltpu.sync_copy(dacc.at[pl.ds(s * RPT, RPT)],
                        deg_out.at[pl.ds(s * RPT, RPT)])


def _agg_body(y, srce, dste, out, sidx, didx, gbuf, acc,
              g0, g1, s0, s1, width, nch):
    c = lax.axis_index("c")
    s = lax.axis_index("s")
    w = s * NC + c
    ch = gbuf.shape[1]
    npairs = nch // 2
    gsem = (g0, g1)
    ssem = (s0, s1)

    def zrow(r, _):
        for k in range(width // 16):
            gbuf[0, r, pl.ds(k * 16, 16)] = jnp.zeros((16,), jnp.float32)
        return 0

    lax.fori_loop(0, ch, zrow, 0)
    for j in range(RPT // ch):
        pltpu.sync_copy(gbuf.at[0], acc.at[pl.ds(s * RPT + j * ch, ch)])
    pltpu.sync_copy(srce.at[w], sidx)
    pltpu.sync_copy(dste.at[w], didx)
    plsc.subcore_barrier()

    def wait_g(j, b):
        pltpu.make_async_copy(y.at[sidx.at[j]], gbuf.at[b], gsem[b]).wait()

    def fire_g(j, b):
        pltpu.async_copy(y.at[sidx.at[j]], gbuf.at[b], gsem[b])

    def wait_s(j, b):
        pltpu.make_async_copy(gbuf.at[b], acc.at[didx.at[j]], ssem[b]).wait()

    def fire_s(j, b):
        pltpu.async_copy(gbuf.at[b], acc.at[didx.at[j]], ssem[b], add=True)

    fire_g(0, 0)

    def grp(p, _):
        j0 = p * 2
        j1 = j0 + 1
        wait_g(j0, 0)
        fire_s(j0, 0)

        @pl.when(p > 0)
        def _():
            wait_s(j0 - 1, 1)

        fire_g(j1, 1)
        wait_g(j1, 1)
        fire_s(j1, 1)
        wait_s(j0, 0)

        @pl.when(p < npairs - 1)
        def _():
            fire_g(j0 + 2, 0)

        return 0

    lax.fori_loop(0, npairs, grp, 0)
    wait_s(nch - 1, 1)
    plsc.subcore_barrier()
    pltpu.sync_copy(acc.at[pl.ds(s * RPT, RPT)],
                    out.at[c, pl.ds(s * RPT, RPT)])


@functools.partial(
    pl.kernel,
    out_type=jax.ShapeDtypeStruct((NC, NPAD, D), jnp.float32),
    mesh=_mesh(),
    compiler_params=_SC_PARAMS,
    scratch_types=[
        pltpu.VMEM((K8, CH8), jnp.int32),
        pltpu.VMEM((K8, CH8), jnp.int32),
        pltpu.VMEM((2, CH8, D), jnp.float32),
        pltpu.VMEM_SHARED((NPAD, D), jnp.float32),
        pltpu.SemaphoreType.DMA,
        pltpu.SemaphoreType.DMA,
        pltpu.SemaphoreType.DMA,
        pltpu.SemaphoreType.DMA,
    ],
)
def _agg128_sc(y1, srce, dste, out, sidx, didx, gbuf, acc, g0, g1, s0, s1):
    _agg_body(y1, srce, dste, out, sidx, didx, gbuf, acc,
              g0, g1, s0, s1, D, K8)


NB16 = 8


@functools.partial(
    pl.kernel,
    out_type=jax.ShapeDtypeStruct((NC, NPAD, DOP), jnp.float32),
    mesh=_mesh(),
    compiler_params=_SC_PARAMS,
    scratch_types=[
        pltpu.VMEM((K16, CH16), jnp.int32),
        pltpu.VMEM((K16, CH16), jnp.int32),
        pltpu.VMEM((NB16, CH16, DOP), jnp.float32),
        pltpu.VMEM_SHARED((NPAD, DOP), jnp.float32),
    ]
    + [pltpu.SemaphoreType.DMA] * (2 * NB16),
)
def _agg16_sc(y2, srce, dste, out, sidx, didx, gbuf, acc, *sems):
    gsem = sems[:NB16]
    ssem = sems[NB16:]
    c = lax.axis_index("c")
    s = lax.axis_index("s")
    w = s * NC + c
    ngrp = K16 // NB16

    def zrow(r, _):
        gbuf[0, r, pl.ds(0, 16)] = jnp.zeros((16,), jnp.float32)
        return 0

    lax.fori_loop(0, CH16, zrow, 0)
    for j in range(RPT // CH16):
        pltpu.sync_copy(gbuf.at[0], acc.at[pl.ds(s * RPT + j * CH16, CH16)])
    pltpu.sync_copy(srce.at[w], sidx)
    pltpu.sync_copy(dste.at[w], didx)
    plsc.subcore_barrier()

    def wait_g(j, b):
        pltpu.make_async_copy(y2.at[sidx.at[j]], gbuf.at[b], gsem[b]).wait()

    def fire_g(j, b):
        pltpu.async_copy(y2.at[sidx.at[j]], gbuf.at[b], gsem[b])

    def wait_s(j, b):
        pltpu.make_async_copy(gbuf.at[b], acc.at[didx.at[j]], ssem[b]).wait()

    def fire_s(j, b):
        pltpu.async_copy(gbuf.at[b], acc.at[didx.at[j]], ssem[b], add=True)

    for b in range(NB16):
        fire_g(b, b)

    def grp(p, _):
        for b in range(NB16):
            j = p * NB16 + b
            wait_g(j, b)
            fire_s(j, b)
        for b in range(NB16):
            j = p * NB16 + b

            @pl.when(p < ngrp - 1)
            def _():
                wait_s(j, b)
                fire_g(j + NB16, b)

        return 0

    lax.fori_loop(0, ngrp, grp, 0)
    for b in range(NB16):
        wait_s((ngrp - 1) * NB16 + b, b)
    plsc.subcore_barrier()
    pltpu.sync_copy(acc.at[pl.ds(s * RPT, RPT)],
                    out.at[c, pl.ds(s * RPT, RPT)])


def _tc1_body(deg_ref, x_ref, w1_ref, y1_ref):
    dis = lax.rsqrt(deg_ref[...] + 1.0)
    xw = jnp.dot(x_ref[...], w1_ref[...], preferred_element_type=jnp.float32)
    y1_ref[...] = xw * dis


def _tc2_body(deg_ref, s1_ref, y1_ref, b1_ref, w2_ref, y2_ref):
    dis = lax.rsqrt(deg_ref[...] + 1.0)
    h = (s1_ref[0] + s1_ref[1] + y1_ref[...]) * dis + b1_ref[...]
    h = jnp.maximum(h, 0.0)
    y2_ref[...] = jnp.dot(h, w2_ref[...], preferred_element_type=jnp.float32) * dis


def _tc3_body(deg_ref, s2_ref, y2_ref, b2_ref, o_ref):
    dis = lax.rsqrt(deg_ref[...] + 1.0)
    o_ref[...] = (s2_ref[0] + s2_ref[1] + y2_ref[...]) * dis + b2_ref[...]


def _pad_edges(col):
    k = EPAD - E
    pad = N + (jnp.arange(k, dtype=jnp.int32) % (NPAD - N))
    return jnp.concatenate([col, pad])


def kernel(x, edge_index, W1, b1, W2, b2):
    ei = edge_index.astype(jnp.int32)
    srcp = _pad_edges(ei[0])
    dstp = _pad_edges(ei[1])
    srce8 = srcp.reshape(NW, K8, CH8)
    dste8 = dstp.reshape(NW, K8, CH8)
    srce16 = srcp.reshape(NW, K16, CH16)
    dste16 = dstp.reshape(NW, K16, CH16)
    xp = jnp.pad(x, ((0, NPAD - N), (0, 0)))
    w2p = jnp.pad(W2, ((0, 0), (0, DOP - DO)))
    b1r = b1.reshape(1, D)
    b2r = jnp.pad(b2, (0, DOP - DO)).reshape(1, DOP)

    deg = _deg_sc(dstp.reshape(NCH16, CH16)).reshape(NPAD, 1)

    y1 = pl.pallas_call(
        _tc1_body,
        grid=(GRID,),
        in_specs=[
            pl.BlockSpec((BLK, 1), lambda i: (i, 0)),
            pl.BlockSpec((BLK, D), lambda i: (i, 0)),
            pl.BlockSpec((D, D), lambda i: (0, 0)),
        ],
        out_specs=pl.BlockSpec((BLK, D), lambda i: (i, 0)),
        out_shape=jax.ShapeDtypeStruct((NPAD, D), jnp.float32),
    )(deg, xp, W1)

    s1 = _agg128_sc(y1, srce8, dste8)

    y2 = pl.pallas_call(
        _tc2_body,
        grid=(GRID,),
        in_specs=[
            pl.BlockSpec((BLK, 1), lambda i: (i, 0)),
            pl.BlockSpec((NC, BLK, D), lambda i: (0, i, 0)),
            pl.BlockSpec((BLK, D), lambda i: (i, 0)),
            pl.BlockSpec((1, D), lambda i: (0, 0)),
            pl.BlockSpec((D, DOP), lambda i: (0, 0)),
        ],
        out_specs=pl.BlockSpec((BLK, DOP), lambda i: (i, 0)),
        out_shape=jax.ShapeDtypeStruct((NPAD, DOP), jnp.float32),
    )(deg, s1, y1, b1r, w2p)

    s2 = _agg16_sc(y2, srce16, dste16)

    o = pl.pallas_call(
        _tc3_body,
        grid=(GRID,),
        in_specs=[
            pl.BlockSpec((BLK, 1), lambda i: (i, 0)),
            pl.BlockSpec((NC, BLK, DOP), lambda i: (0, i, 0)),
            pl.BlockSpec((BLK, DOP), lambda i: (i, 0)),
            pl.BlockSpec((1, DOP), lambda i: (0, 0)),
        ],
        out_specs=pl.BlockSpec((BLK, DOP), lambda i: (i, 0)),
        out_shape=jax.ShapeDtypeStruct((NPAD, DOP), jnp.float32),
    )(deg, s2, y2, b2r)

    return o[:N, :DO]


# speedup vs baseline: 4.0979x; 1.1768x over previous
import functools
import jax
import jax.numpy as jnp
from jax import lax
from jax.experimental import pallas as pl
from jax.experimental.pallas import tpu as pltpu
from jax.experimental.pallas import tpu_sc as plsc

N = 10000
D = 128
DO = 2
DOP = 16
E = 320000
NPAD = 10240
NC = 2
NS = 16
NW = NC * NS
RPT = NPAD // NS
BLK = 1024
GRID = NPAD // BLK
EPAD = 327680

CH8 = 80
K8 = 128
CH16 = 128
K16 = 80
NCH16 = 2560

_SC_PARAMS = pltpu.CompilerParams(use_tc_tiling_on_sc=False)


def _mesh():
    return plsc.VectorSubcoreMesh(core_axis_name="c", subcore_axis_name="s")


@functools.partial(
    pl.kernel,
    out_type=jax.ShapeDtypeStruct((NPAD,), jnp.float32),
    mesh=_mesh(),
    compiler_params=_SC_PARAMS,
    scratch_types=[
        pltpu.VMEM((NCH16 // NS, CH16), jnp.int32),
        pltpu.VMEM((CH16,), jnp.float32),
        pltpu.VMEM((RPT,), jnp.float32),
        pltpu.VMEM_SHARED((NPAD,), jnp.float32),
        pltpu.SemaphoreType.DMA,
    ],
)
def _deg_sc(dste, deg_out, didx, ones, zbuf, dacc, dsem):
    c = lax.axis_index("c")
    s = lax.axis_index("s")
    nch = NCH16 // NS
    for k in range(CH16 // 16):
        ones[pl.ds(k * 16, 16)] = jnp.ones((16,), jnp.float32)
    for k in range(RPT // 16):
        zbuf[pl.ds(k * 16, 16)] = jnp.zeros((16,), jnp.float32)

    @pl.when(c == 0)
    def _():
        pltpu.sync_copy(zbuf, dacc.at[pl.ds(s * RPT, RPT)])
        plsc.subcore_barrier()
        pltpu.sync_copy(dste.at[pl.ds(s * nch, nch)], didx)

        def body(p, _):
            for b in range(8):
                pltpu.async_copy(ones, dacc.at[didx.at[p * 8 + b]],
                                 dsem, add=True)
            for b in range(8):
                pltpu.make_async_copy(ones, dacc.at[didx.at[p * 8 + b]],
                                      dsem).wait()
            return 0

        lax.fori_loop(0, nch // 8, body, 0)
        plsc.subcore_barrier()
        pltpu.sync_copy(dacc.at[pl.ds(s * RPT, RPT)],
                        deg_out.at[pl.ds(s * RPT, RPT)])


NB8 = 4
NPH8 = 4
CPP8 = K8 // NPH8


@functools.partial(
    pl.kernel,
    out_type=jax.ShapeDtypeStruct((NC, NPAD, D), jnp.float32),
    mesh=_mesh(),
    compiler_params=_SC_PARAMS,
    scratch_types=[
        pltpu.VMEM((CPP8, CH8), jnp.int32),
        pltpu.VMEM((CPP8, CH8), jnp.int32),
        pltpu.VMEM((NB8, CH8, D), jnp.float32),
        pltpu.VMEM_SHARED((NPAD, D), jnp.float32),
    ]
    + [pltpu.SemaphoreType.DMA] * (2 * NB8),
)
def _agg128_sc(y1, srce, dste, out, sidx, didx, gbuf, acc, *sems):
    gsem = sems[:NB8]
    ssem = sems[NB8:]
    c = lax.axis_index("c")
    s = lax.axis_index("s")
    w = s * NC + c
    ngrp = CPP8 // NB8

    def zrow(r, _):
        for k in range(D // 16):
            gbuf[0, r, pl.ds(k * 16, 16)] = jnp.zeros((16,), jnp.float32)
        return 0

    lax.fori_loop(0, CH8, zrow, 0)
    for j in range(RPT // CH8):
        pltpu.sync_copy(gbuf.at[0], acc.at[pl.ds(s * RPT + j * CH8, CH8)])
    plsc.subcore_barrier()

    def wait_g(j, b):
        pltpu.make_async_copy(y1.at[sidx.at[j]], gbuf.at[b], gsem[b]).wait()

    def fire_g(j, b):
        pltpu.async_copy(y1.at[sidx.at[j]], gbuf.at[b], gsem[b])

    def wait_s(j, b):
        pltpu.make_async_copy(gbuf.at[b], acc.at[didx.at[j]], ssem[b]).wait()

    def fire_s(j, b):
        pltpu.async_copy(gbuf.at[b], acc.at[didx.at[j]], ssem[b], add=True)

    for ph in range(NPH8):
        pltpu.sync_copy(srce.at[w, pl.ds(ph * CPP8, CPP8)], sidx)
        pltpu.sync_copy(dste.at[w, pl.ds(ph * CPP8, CPP8)], didx)
        for b in range(NB8):
            fire_g(b, b)

        def grp(p, _):
            for b in range(NB8):
                j = p * NB8 + b
                wait_g(j, b)
                fire_s(j, b)
            for b in range(NB8):
                j = p * NB8 + b

                @pl.when(p < ngrp - 1)
                def _():
                    wait_s(j, b)
                    fire_g(j + NB8, b)

            return 0

        lax.fori_loop(0, ngrp, grp, 0)
        for b in range(NB8):
            wait_s((ngrp - 1) * NB8 + b, b)
    plsc.subcore_barrier()
    pltpu.sync_copy(acc.at[pl.ds(s * RPT, RPT)],
                    out.at[c, pl.ds(s * RPT, RPT)])


NB16 = 8


@functools.partial(
    pl.kernel,
    out_type=jax.ShapeDtypeStruct((NC, NPAD, DOP), jnp.float32),
    mesh=_mesh(),
    compiler_params=_SC_PARAMS,
    scratch_types=[
        pltpu.VMEM((K16, CH16), jnp.int32),
        pltpu.VMEM((K16, CH16), jnp.int32),
        pltpu.VMEM((NB16, CH16, DOP), jnp.float32),
        pltpu.VMEM_SHARED((NPAD, DOP), jnp.float32),
    ]
    + [pltpu.SemaphoreType.DMA] * (2 * NB16),
)
def _agg16_sc(y2, srce, dste, out, sidx, didx, gbuf, acc, *sems):
    gsem = sems[:NB16]
    ssem = sems[NB16:]
    c = lax.axis_index("c")
    s = lax.axis_index("s")
    w = s * NC + c
    ngrp = K16 // NB16

    def zrow(r, _):
        gbuf[0, r, pl.ds(0, 16)] = jnp.zeros((16,), jnp.float32)
        return 0

    lax.fori_loop(0, CH16, zrow, 0)
    for j in range(RPT // CH16):
        pltpu.sync_copy(gbuf.at[0], acc.at[pl.ds(s * RPT + j * CH16, CH16)])
    pltpu.sync_copy(srce.at[w], sidx)
    pltpu.sync_copy(dste.at[w], didx)
    plsc.subcore_barrier()

    def wait_g(j, b):
        pltpu.make_async_copy(y2.at[sidx.at[j]], gbuf.at[b], gsem[b]).wait()

    def fire_g(j, b):
        pltpu.async_copy(y2.at[sidx.at[j]], gbuf.at[b], gsem[b])

    def wait_s(j, b):
        pltpu.make_async_copy(gbuf.at[b], acc.at[didx.at[j]], ssem[b]).wait()

    def fire_s(j, b):
        pltpu.async_copy(gbuf.at[b], acc.at[didx.at[j]], ssem[b], add=True)

    for b in range(NB16):
        fire_g(b, b)

    def grp(p, _):
        for b in range(NB16):
            j = p * NB16 + b
            wait_g(j, b)
            fire_s(j, b)
        for b in range(NB16):
            j = p * NB16 + b

            @pl.when(p < ngrp - 1)
            def _():
                wait_s(j, b)
                fire_g(j + NB16, b)

        return 0

    lax.fori_loop(0, ngrp, grp, 0)
    for b in range(NB16):
        wait_s((ngrp - 1) * NB16 + b, b)
    plsc.subcore_barrier()
    pltpu.sync_copy(acc.at[pl.ds(s * RPT, RPT)],
                    out.at[c, pl.ds(s * RPT, RPT)])


def _tc1_body(deg_ref, x_ref, w1_ref, y1_ref):
    dis = lax.rsqrt(deg_ref[...] + 1.0)
    xw = jnp.dot(x_ref[...], w1_ref[...], preferred_element_type=jnp.float32)
    y1_ref[...] = xw * dis


def _tc2_body(deg_ref, s1_ref, y1_ref, b1_ref, w2_ref, y2_ref):
    dis = lax.rsqrt(deg_ref[...] + 1.0)
    h = (s1_ref[0] + s1_ref[1] + y1_ref[...]) * dis + b1_ref[...]
    h = jnp.maximum(h, 0.0)
    y2_ref[...] = jnp.dot(h, w2_ref[...], preferred_element_type=jnp.float32) * dis


def _tc3_body(deg_ref, s2_ref, y2_ref, b2_ref, o_ref):
    dis = lax.rsqrt(deg_ref[...] + 1.0)
    o_ref[...] = (s2_ref[0] + s2_ref[1] + y2_ref[...]) * dis + b2_ref[...]


def _pad_edges(col):
    k = EPAD - E
    pad = N + (jnp.arange(k, dtype=jnp.int32) % (NPAD - N))
    return jnp.concatenate([col, pad])


def kernel(x, edge_index, W1, b1, W2, b2):
    ei = edge_index.astype(jnp.int32)
    srcp = _pad_edges(ei[0])
    dstp = _pad_edges(ei[1])
    srce8 = srcp.reshape(NW, K8, CH8)
    dste8 = dstp.reshape(NW, K8, CH8)
    srce16 = srcp.reshape(NW, K16, CH16)
    dste16 = dstp.reshape(NW, K16, CH16)
    xp = jnp.pad(x, ((0, NPAD - N), (0, 0)))
    w2p = jnp.pad(W2, ((0, 0), (0, DOP - DO)))
    b1r = b1.reshape(1, D)
    b2r = jnp.pad(b2, (0, DOP - DO)).reshape(1, DOP)

    deg = _deg_sc(dstp.reshape(NCH16, CH16)).reshape(NPAD, 1)

    y1 = pl.pallas_call(
        _tc1_body,
        grid=(GRID,),
        in_specs=[
            pl.BlockSpec((BLK, 1), lambda i: (i, 0)),
            pl.BlockSpec((BLK, D), lambda i: (i, 0)),
            pl.BlockSpec((D, D), lambda i: (0, 0)),
        ],
        out_specs=pl.BlockSpec((BLK, D), lambda i: (i, 0)),
        out_shape=jax.ShapeDtypeStruct((NPAD, D), jnp.float32),
    )(deg, xp, W1)

    s1 = _agg128_sc(y1, srce8, dste8)

    y2 = pl.pallas_call(
        _tc2_body,
        grid=(GRID,),
        in_specs=[
            pl.BlockSpec((BLK, 1), lambda i: (i, 0)),
            pl.BlockSpec((NC, BLK, D), lambda i: (0, i, 0)),
            pl.BlockSpec((BLK, D), lambda i: (i, 0)),
            pl.BlockSpec((1, D), lambda i: (0, 0)),
            pl.BlockSpec((D, DOP), lambda i: (0, 0)),
        ],
        out_specs=pl.BlockSpec((BLK, DOP), lambda i: (i, 0)),
        out_shape=jax.ShapeDtypeStruct((NPAD, DOP), jnp.float32),
    )(deg, s1, y1, b1r, w2p)

    s2 = _agg16_sc(y2, srce16, dste16)

    o = pl.pallas_call(
        _tc3_body,
        grid=(GRID,),
        in_specs=[
            pl.BlockSpec((BLK, 1), lambda i: (i, 0)),
            pl.BlockSpec((NC, BLK, DOP), lambda i: (0, i, 0)),
            pl.BlockSpec((BLK, DOP), lambda i: (i, 0)),
            pl.BlockSpec((1, DOP), lambda i: (0, 0)),
        ],
        out_specs=pl.BlockSpec((BLK, DOP), lambda i: (i, 0)),
        out_shape=jax.ShapeDtypeStruct((NPAD, DOP), jnp.float32),
    )(deg, s2, y2, b2r)

    return o[:N, :DO]
